# Initial kernel scaffold; baseline (speedup 1.0000x reference)
import jax
import jax.numpy as jnp
from jax.experimental import pallas as pl


def kernel(z, W):
    raise NotImplementedError("write your pallas kernel here")


# speedup vs baseline: 1.0628x; 1.0628x over previous
import functools

import jax
import jax.numpy as jnp
from jax import lax
from jax.experimental import pallas as pl
from jax.experimental.pallas import tpu as pltpu
from jax.experimental.pallas import tpu_sc as plsc

N_CODES = 8192
D_EMB = 32
COMMIT_BETA = 0.25

RB1 = 512
CB1 = 4096

RB2 = 512
CB2 = 2048

SC_CORES = 2
SC_SUBCORES = 16
SC_WORKERS = SC_CORES * SC_SUBCORES


def _argmin_body(zsq_ref, z_ref, wt_ref, wsq_ref, idx_ref, best_ref, bidx_ref):
    ct = pl.program_id(1)
    zb = z_ref[...].astype(jnp.bfloat16)
    wb = wt_ref[...].astype(jnp.bfloat16)
    mm = lax.dot_general(
        zb, wb, (((1,), (0,)), ((), ())),
        preferred_element_type=jnp.float32)
    d = (zsq_ref[...] + wsq_ref[...]) - 2.0 * mm
    m = jnp.min(d, axis=1, keepdims=True)
    cols = lax.broadcasted_iota(jnp.int32, d.shape, 1) + ct * CB1
    cand = jnp.where(d == m, cols, jnp.int32(2 ** 30))
    am = jnp.min(cand, axis=1, keepdims=True)

    @pl.when(ct == 0)
    def _():
        best_ref[...] = m
        bidx_ref[...] = am

    @pl.when(ct == 1)
    def _():
        m0q = best_ref[...].astype(jnp.bfloat16).astype(jnp.float32)
        idx_ref[...] = jnp.where(m < m0q, am, bidx_ref[...])


def _argmin_call(zsq, zf, wt, wsq):
    n = zf.shape[0]
    grid = (n // RB1, N_CODES // CB1)
    return pl.pallas_call(
        _argmin_body,
        grid=grid,
        in_specs=[
            pl.BlockSpec((RB1, 1), lambda rt, ct: (rt, 0)),
            pl.BlockSpec((RB1, D_EMB), lambda rt, ct: (rt, 0)),
            pl.BlockSpec((D_EMB, CB1), lambda rt, ct: (0, ct)),
            pl.BlockSpec((1, CB1), lambda rt, ct: (0, ct)),
        ],
        out_specs=pl.BlockSpec((RB1, 1), lambda rt, ct: (rt, 0)),
        out_shape=jax.ShapeDtypeStruct((n, 1), jnp.int32),
        scratch_shapes=[
            pltpu.VMEM((RB1, 1), jnp.float32),
            pltpu.VMEM((RB1, 1), jnp.int32),
        ],
    )(zsq, zf, wt, wsq)


def _onehot_body(idx_ref, enc_ref, cnt_ref):
    ct = pl.program_id(0)
    rt = pl.program_id(1)
    cols = lax.broadcasted_iota(jnp.int32, (RB2, CB2), 1) + ct * CB2
    oh = (idx_ref[...] == cols).astype(jnp.float32)
    enc_ref[...] = oh

    @pl.when(rt == 0)
    def _():
        cnt_ref[...] = jnp.zeros_like(cnt_ref)

    cnt_ref[...] += jnp.sum(oh, axis=0, keepdims=True)


def _onehot_call(idx2, n):
    grid = (N_CODES // CB2, n // RB2)
    return pl.pallas_call(
        _onehot_body,
        grid=grid,
        in_specs=[
            pl.BlockSpec((RB2, 1), lambda ct, rt: (rt, 0)),
        ],
        out_specs=[
            pl.BlockSpec((RB2, CB2), lambda ct, rt: (rt, ct)),
            pl.BlockSpec((1, CB2), lambda ct, rt: (0, ct)),
        ],
        out_shape=[
            jax.ShapeDtypeStruct((n, N_CODES), jnp.float32),
            jax.ShapeDtypeStruct((1, N_CODES), jnp.float32),
        ],
    )(idx2)


def _make_gather(n):
    rows_per_w = n // SC_WORKERS
    idx_rows = rows_per_w // 128
    mesh = plsc.VectorSubcoreMesh(core_axis_name="c", subcore_axis_name="s")

    @functools.partial(
        pl.kernel,
        mesh=mesh,
        compiler_params=pltpu.CompilerParams(use_tc_tiling_on_sc=False),
        out_type=jax.ShapeDtypeStruct((n, D_EMB), jnp.float32),
        scratch_types=[
            pltpu.VMEM((idx_rows, 128), jnp.int32),
            pltpu.VMEM((rows_per_w, D_EMB), jnp.float32),
            pltpu.SemaphoreType.DMA,
        ],
    )
    def _gather(w_hbm, idx_hbm, out_hbm, idx_v, rows_v, sem):
        wid = lax.axis_index("s") * SC_CORES + lax.axis_index("c")
        pltpu.sync_copy(idx_hbm.at[pl.ds(wid * idx_rows, idx_rows)], idx_v)
        copies = []
        for j in range(idx_rows):
            copies.append(pltpu.async_copy(
                w_hbm.at[idx_v.at[j]],
                rows_v.at[pl.ds(j * 128, 128)],
                sem))
        for c in copies:
            c.wait()
        pltpu.sync_copy(rows_v, out_hbm.at[pl.ds(wid * rows_per_w, rows_per_w)])

    return _gather


def kernel(z, W):
    batch, seq, d_emb = z.shape
    zf = z.reshape(-1, d_emb)
    n = zf.shape[0]

    zsq = jnp.sum(z ** 2, axis=2).reshape(-1, 1)
    wsq = jnp.sum(W ** 2, axis=1)

    idx2 = _argmin_call(zsq, zf, W.T, wsq.reshape(1, -1))
    enc, cnt = _onehot_call(idx2, n)

    idx_flat = idx2[:, 0]
    zq_flat = _make_gather(n)(W, idx_flat.reshape(-1, 128))
    z_q = zq_flat.astype(jnp.bfloat16).astype(jnp.float32).reshape(batch, seq, d_emb)

    loss = (jnp.mean((lax.stop_gradient(z_q) - z) ** 2)
            + COMMIT_BETA * jnp.mean((z_q - lax.stop_gradient(z)) ** 2))
    z_q_st = z + lax.stop_gradient(z_q - z)
    e_mean = cnt[0] / jnp.float32(n)
    perplexity = jnp.exp(-jnp.sum(e_mean * jnp.log(e_mean + 1e-10)))
    idx_out = idx_flat.reshape(batch, seq)
    return (loss, z_q_st, perplexity, enc, idx_out)
